# baseline (speedup 1.0000x reference)
import functools

import jax
import jax.numpy as jnp
from jax import lax
from jax.experimental import pallas as pl
from jax.experimental.pallas import tpu as pltpu
from jax.experimental.pallas import tpu_sc as plsc

N = 10000
E = 160000
D_IN = 256
NP = 10240
NR = NP // 16
NC, NS, L = 2, 16, 16
B = 80
EPT = E // NS
NB = EPT // B
f32 = jnp.float32
i32 = jnp.int32

_SC_PARAMS = pltpu.CompilerParams(use_tc_tiling_on_sc=False,
                                  needs_layout_passes=False)


RB = 1000


RE = E // (N // RB)


def _tc1_body(x_r, w_r, as_r, ad_r, ea_r, ae_r, h_r, hs_r, hd_r, eo_r):
    i = pl.program_id(0)
    hb = jnp.dot(x_r[...], w_r[...], preferred_element_type=f32)
    h_r[...] = hb
    hs_r[...] = jnp.dot(hb, as_r[...], preferred_element_type=f32)
    hd_r[...] = jnp.dot(hb, ad_r[...], preferred_element_type=f32)
    eo_r[pl.ds(i * RE, RE)] = jnp.dot(ea_r[...], ae_r[...],
                                      preferred_element_type=f32)[:, 0]


def _tc1(x, attW, a_src2, a_dst2, edge_attr, a_edge2):
    return pl.pallas_call(
        _tc1_body,
        grid=(N // RB,),
        in_specs=[
            pl.BlockSpec((RB, D_IN), lambda i: (i, 0)),
            pl.BlockSpec((D_IN, D_IN), lambda i: (0, 0)),
            pl.BlockSpec((D_IN, 1), lambda i: (0, 0)),
            pl.BlockSpec((D_IN, 1), lambda i: (0, 0)),
            pl.BlockSpec((RE, 16), lambda i: (i, 0)),
            pl.BlockSpec((16, 1), lambda i: (0, 0)),
        ],
        out_specs=[
            pl.BlockSpec((RB, D_IN), lambda i: (i, 0)),
            pl.BlockSpec((RB, 1), lambda i: (i, 0)),
            pl.BlockSpec((RB, 1), lambda i: (i, 0)),
            pl.BlockSpec((E,), lambda i: (0,)),
        ],
        out_shape=[
            jax.ShapeDtypeStruct((N, D_IN), f32),
            jax.ShapeDtypeStruct((N, 1), f32),
            jax.ShapeDtypeStruct((N, 1), f32),
            jax.ShapeDtypeStruct((E,), f32),
        ],
    )(x, attW, a_src2, a_dst2, edge_attr, a_edge2)


def _tc2_body(o0_r, o1_r, o2_r, o3_r, den_r, deg_r, gw_r, hg_r, hgs_r, dinv_r):
    rden = 1.0 / (den_r[...] + 1e-16)
    x1 = jnp.concatenate(
        [jnp.maximum(o0_r[...] * rden, 0.0), jnp.maximum(o1_r[...] * rden, 0.0),
         jnp.maximum(o2_r[...] * rden, 0.0), jnp.maximum(o3_r[...] * rden, 0.0)],
        axis=1)
    acc = jnp.dot(x1, gw_r[...], preferred_element_type=f32)
    hg_r[...] = acc
    dv = lax.rsqrt(deg_r[...] + 1.0)
    dinv_r[...] = dv
    hgs_r[...] = dv * acc


def _tc2(o1q, den, deg, gcnW):
    return pl.pallas_call(
        _tc2_body,
        grid=(N // RB,),
        in_specs=[
            pl.BlockSpec((RB, 64), lambda i: (i, 0)),
            pl.BlockSpec((RB, 64), lambda i: (i, 0)),
            pl.BlockSpec((RB, 64), lambda i: (i, 0)),
            pl.BlockSpec((RB, 64), lambda i: (i, 0)),
            pl.BlockSpec((RB, 1), lambda i: (i, 0)),
            pl.BlockSpec((RB, 1), lambda i: (i, 0)),
            pl.BlockSpec((D_IN, 128), lambda i: (0, 0)),
        ],
        out_specs=[
            pl.BlockSpec((RB, 128), lambda i: (i, 0)),
            pl.BlockSpec((RB, 128), lambda i: (i, 0)),
            pl.BlockSpec((RB, 1), lambda i: (i, 0)),
        ],
        out_shape=[
            jax.ShapeDtypeStruct((N, 128), f32),
            jax.ShapeDtypeStruct((N, 128), f32),
            jax.ShapeDtypeStruct((N, 1), f32),
        ],
    )(o1q[0], o1q[1], o1q[2], o1q[3], den, deg, gcnW)


def _tc3_body(p0_r, p1_r, p2_r, p3_r, hg_r, dinv_r, goal_r, gcnb_r,
              w1a_r, w1b_r, db1_r, w2_r, db2_r, pred_r, x2_r):
    dv = dinv_r[...]
    sc_part = jnp.concatenate(
        [p0_r[...], p1_r[...], p2_r[...], p3_r[...]], axis=1)
    out2 = dv * sc_part + (dv * dv) * hg_r[...] + gcnb_r[...]
    x2 = jnp.maximum(out2, 0.0)
    x2_r[...] = x2
    hdist = (jnp.dot(x2, w1a_r[...], preferred_element_type=f32)
             + jnp.dot(goal_r[...], w1b_r[...], preferred_element_type=f32)
             + db1_r[...])
    hdist = jnp.maximum(hdist, 0.0)
    pred_r[...] = jnp.dot(hdist, w2_r[...], preferred_element_type=f32) + db2_r[...]


def _tc3(p, hg, dinv, goal, gcnb2, w1a, w1b, db1_2, dW2, db2_2):
    return pl.pallas_call(
        _tc3_body,
        grid=(N // RB,),
        in_specs=[
            pl.BlockSpec((RB, 32), lambda i: (i, 0)),
            pl.BlockSpec((RB, 32), lambda i: (i, 0)),
            pl.BlockSpec((RB, 32), lambda i: (i, 0)),
            pl.BlockSpec((RB, 32), lambda i: (i, 0)),
            pl.BlockSpec((RB, 128), lambda i: (i, 0)),
            pl.BlockSpec((RB, 1), lambda i: (i, 0)),
            pl.BlockSpec((RB, D_IN), lambda i: (i, 0)),
            pl.BlockSpec((1, 128), lambda i: (0, 0)),
            pl.BlockSpec((128, D_IN), lambda i: (0, 0)),
            pl.BlockSpec((D_IN, D_IN), lambda i: (0, 0)),
            pl.BlockSpec((1, D_IN), lambda i: (0, 0)),
            pl.BlockSpec((D_IN, 1), lambda i: (0, 0)),
            pl.BlockSpec((1, 1), lambda i: (0, 0)),
        ],
        out_specs=[
            pl.BlockSpec((RB, 1), lambda i: (i, 0)),
            pl.BlockSpec((RB, 128), lambda i: (i, 0)),
        ],
        out_shape=[
            jax.ShapeDtypeStruct((N, 1), f32),
            jax.ShapeDtypeStruct((N, 128), f32),
        ],
    )(p[0], p[1], p[2], p[3], hg, dinv, goal, gcnb2, w1a, w1b, db1_2,
      dW2, db2_2)


def _zero_2d(ref, nrows):
    z = jnp.zeros((L,), f32)

    def body(r, _):
        ref[r] = z
        return 0
    lax.fori_loop(0, nrows, body, 0)


def _zero_buf(buf, w):
    z = jnp.zeros((L,), f32)

    def body(b, _):
        for f in range(w):
            buf[b, pl.ds(f * L, L)] = z
        return 0
    lax.fori_loop(0, B, body, 0)


def _scale_rows(buf, w, wref, woff):
    def body(b, _):
        spl = plsc.load_gather(wref, [jnp.full((L,), woff + b, i32)])
        for f in range(w):
            sl = pl.ds(f * L, L)
            buf[b, sl] = buf[b, sl] * spl
        return 0
    lax.fori_loop(0, B, body, 0)


def _iota_fill(ixb, base):
    io = lax.iota(i32, L)

    def body(j, _):
        ixb[pl.ds(j * L, L)] = io + (base + j * L)
        return 0
    lax.fori_loop(0, B // L, body, 0)


def _tile_reduce(part_v, sh, ixb):
    def body(j, _):
        _iota_fill(ixb, j * B)
        pltpu.sync_copy(part_v.at[pl.ds(j * B, B)], sh.at[ixb], add=True)
        return 0
    lax.fori_loop(0, NR // B, body, 0)


def _spmm_pass(table_h, tq, src_t, wfill, wref, woff_fn, acc_sh, w,
               hbufs, ixs, dbs, gsem, ssem):
    def issue_gather(b, cur):
        off = b * B

        def wl(j, _):
            sl = pl.ds(j * L, L)
            ixs[cur][sl] = src_t[pl.ds(off + j * L, L)] * 4 + tq
            return 0
        lax.fori_loop(0, B // L, wl, 0)
        pltpu.async_copy(table_h.at[ixs[cur]], hbufs[cur], gsem[cur])

    def wait_gather(cur):
        pltpu.make_async_copy(table_h.at[ixs[cur]], hbufs[cur],
                              gsem[cur]).wait()

    def issue_scatter(b, cur):
        wfill(b, cur)
        if wref is not None:
            _scale_rows(hbufs[cur], w, wref, woff_fn(b))
        pltpu.async_copy(hbufs[cur], acc_sh.at[dbs[cur]], ssem[cur], add=True)

    def wait_scatter(cur):
        pltpu.make_async_copy(hbufs[cur], acc_sh.at[dbs[cur]], ssem[cur]).wait()

    issue_gather(0, 0)
    issue_gather(1, 1)
    wait_gather(0)
    issue_gather(2, 2)
    issue_scatter(0, 0)
    wait_gather(1)
    issue_scatter(1, 1)

    def group(g, _):
        b = g * 3 + 2
        for j, cur in enumerate((2, 0, 1)):
            nxt = (cur + 1) % 3
            wait_scatter(nxt)
            issue_gather(b + j + 1, nxt)
            wait_gather(cur)
            issue_scatter(b + j, cur)
        return 0
    lax.fori_loop(0, (NB - 5) // 3, group, 0)

    wait_scatter(0)
    issue_gather(NB - 2, 0)
    wait_gather(2)
    issue_scatter(NB - 3, 2)
    wait_scatter(1)
    issue_gather(NB - 1, 1)
    wait_gather(0)
    issue_scatter(NB - 2, 0)
    wait_gather(1)
    issue_scatter(NB - 1, 1)
    wait_scatter(2)
    wait_scatter(0)
    wait_scatter(1)


def _sc1_body(h4_h, hs_h, hd_h, ea_h, src_h, dst_h,
              o0_h, o1_h, o2_h, o3_h, deg_h, den_h,
              src_t, dst_t, ea_t, hs_v, hd_v, den_v, deg_v, zbuf,
              hb0, hb1, hb2, ix0, ix1, ix2, db0, db1_, db2_,
              gs0, gs1, gs2, ss0, ss1, ss2,
              acc_sh, den_sh, deg_sh):
    c = lax.axis_index("c")
    s = lax.axis_index("s")
    hbufs = [hb0, hb1, hb2]
    ixs = [ix0, ix1, ix2]
    dbs = [db0, db1_, db2_]
    gsem = [gs0, gs1, gs2]
    ssem = [ss0, ss1, ss2]
    o_hs = [o0_h, o1_h, o2_h, o3_h]
    base = s * EPT

    pltpu.sync_copy(src_h.at[pl.ds(base, EPT)], src_t)
    pltpu.sync_copy(dst_h.at[pl.ds(base, EPT)], dst_t)
    pltpu.sync_copy(ea_h.at[pl.ds(base, EPT)], ea_t)
    pltpu.sync_copy(hs_h, hs_v)
    pltpu.sync_copy(hd_h, hd_v)

    _zero_2d(den_v, NR)
    _zero_2d(deg_v, NR)
    _zero_2d(zbuf, NR // NS)

    srows = pl.ds(s * (NR // NS), NR // NS)
    pltpu.sync_copy(zbuf, den_sh.at[srows])
    pltpu.sync_copy(zbuf, deg_sh.at[srows])
    _zero_buf(hb0, 4)

    def zacc(j, _):
        pltpu.sync_copy(hb0, acc_sh.at[pl.ds(s * 640 + j * B, B)])
        return 0
    lax.fori_loop(0, 640 // B, zacc, 0)

    ones = jnp.ones((L,), f32)

    def p1(k, _):
        off = k * L
        s16 = src_t[pl.ds(off, L)]
        d16 = dst_t[pl.ds(off, L)]
        z = (plsc.load_gather(hs_v, [s16]) + plsc.load_gather(hd_v, [d16])
             + ea_t[pl.ds(off, L)])
        zl = jnp.where(z >= 0.0, z, 0.2 * z)
        e16 = jnp.exp(zl)
        ea_t[pl.ds(off, L)] = e16
        dr = lax.shift_right_logical(d16, 4)
        dc = lax.bitwise_and(d16, 15)
        plsc.addupdate_scatter(den_v, [dr, dc], e16)
        plsc.addupdate_scatter(deg_v, [dr, dc], ones)
        return 0
    lax.fori_loop(0, EPT // L, p1, 0)

    plsc.subcore_barrier()
    _tile_reduce(den_v, den_sh, ix0)
    _tile_reduce(deg_v, deg_sh, ix0)
    plsc.subcore_barrier()

    @pl.when(c == 0)
    def _():
        pltpu.sync_copy(deg_sh.at[srows], deg_h.at[srows])
        pltpu.sync_copy(den_sh.at[srows], den_h.at[srows])

    def wfill(b, cur):
        off = b * B

        def w(j, _):
            sl = pl.ds(j * L, L)
            dbs[cur][sl] = dst_t[pl.ds(off + j * L, L)]
            return 0
        lax.fori_loop(0, B // L, w, 0)

    for qi in range(2):
        tq = 2 * c + qi
        _spmm_pass(h4_h, tq, src_t, wfill, ea_t, lambda b: b * B,
                   acc_sh, 4, hbufs, ixs, dbs, gsem, ssem)
        plsc.subcore_barrier()
        rows = pl.ds(s * 640, 640)

        @pl.when(c == 0)
        def _():
            pltpu.sync_copy(acc_sh.at[rows], o_hs[qi].at[rows])

        @pl.when(c == 1)
        def _():
            pltpu.sync_copy(acc_sh.at[rows], o_hs[2 + qi].at[rows])

        if qi == 0:
            _zero_buf(hb0, 4)

            def zacc2(j, _):
                pltpu.sync_copy(hb0, acc_sh.at[pl.ds(s * 640 + j * B, B)])
                return 0
            lax.fori_loop(0, 640 // B, zacc2, 0)
            plsc.subcore_barrier()


def _sc1(h4, hs, hd, eatt, src, dst):
    mesh = plsc.VectorSubcoreMesh(core_axis_name="c", subcore_axis_name="s")
    f = functools.partial(
        pl.kernel,
        out_type=[
            jax.ShapeDtypeStruct((NP, 64), f32),
            jax.ShapeDtypeStruct((NP, 64), f32),
            jax.ShapeDtypeStruct((NP, 64), f32),
            jax.ShapeDtypeStruct((NP, 64), f32),
            jax.ShapeDtypeStruct((NR, 16), f32),
            jax.ShapeDtypeStruct((NR, 16), f32),
        ],
        mesh=mesh,
        scratch_types=[
            pltpu.VMEM((EPT,), i32),
            pltpu.VMEM((EPT,), i32),
            pltpu.VMEM((EPT,), f32),
            pltpu.VMEM((N,), f32),
            pltpu.VMEM((N,), f32),
            pltpu.VMEM((NR, 16), f32),
            pltpu.VMEM((NR, 16), f32),
            pltpu.VMEM((NR // NS, 16), f32),
            pltpu.VMEM((B, 64), f32),
            pltpu.VMEM((B, 64), f32),
            pltpu.VMEM((B, 64), f32),
            pltpu.VMEM((B,), i32),
            pltpu.VMEM((B,), i32),
            pltpu.VMEM((B,), i32),
            pltpu.VMEM((B,), i32),
            pltpu.VMEM((B,), i32),
            pltpu.VMEM((B,), i32),
            pltpu.SemaphoreType.DMA,
            pltpu.SemaphoreType.DMA,
            pltpu.SemaphoreType.DMA,
            pltpu.SemaphoreType.DMA,
            pltpu.SemaphoreType.DMA,
            pltpu.SemaphoreType.DMA,
            pltpu.VMEM_SHARED((NP, 64), f32),
            pltpu.VMEM_SHARED((NR, 16), f32),
            pltpu.VMEM_SHARED((NR, 16), f32),
        ],
        compiler_params=_SC_PARAMS,
    )(_sc1_body)
    return f(h4, hs, hd, eatt, src, dst)


def _sc2_body(hg4_h, src_h, dst_h,
              o0_h, o1_h, o2_h, o3_h,
              src_t, dst_t,
              hb0, hb1, hb2, ix0, ix1, ix2, db0, db1_, db2_,
              gs0, gs1, gs2, ss0, ss1, ss2,
              acc_sh):
    c = lax.axis_index("c")
    s = lax.axis_index("s")
    hbufs = [hb0, hb1, hb2]
    ixs = [ix0, ix1, ix2]
    dbs = [db0, db1_, db2_]
    gsem = [gs0, gs1, gs2]
    ssem = [ss0, ss1, ss2]
    o_hs = [o0_h, o1_h, o2_h, o3_h]
    base = s * EPT

    pltpu.sync_copy(src_h.at[pl.ds(base, EPT)], src_t)
    pltpu.sync_copy(dst_h.at[pl.ds(base, EPT)], dst_t)

    _zero_buf(hb0, 2)

    def zacc(j, _):
        pltpu.sync_copy(hb0, acc_sh.at[pl.ds(s * 640 + j * B, B)])
        return 0
    lax.fori_loop(0, 640 // B, zacc, 0)
    plsc.subcore_barrier()

    def wfill(b, cur):
        off = b * B

        def w(j, _):
            sl = pl.ds(j * L, L)
            dbs[cur][sl] = dst_t[pl.ds(off + j * L, L)]
            return 0
        lax.fori_loop(0, B // L, w, 0)

    for qi in range(2):
        tq = 2 * c + qi
        _spmm_pass(hg4_h, tq, src_t, wfill, None, lambda b: b * B,
                   acc_sh, 2, hbufs, ixs, dbs, gsem, ssem)
        plsc.subcore_barrier()
        rows = pl.ds(s * 640, 640)

        @pl.when(c == 0)
        def _():
            pltpu.sync_copy(acc_sh.at[rows], o_hs[qi].at[rows])

        @pl.when(c == 1)
        def _():
            pltpu.sync_copy(acc_sh.at[rows], o_hs[2 + qi].at[rows])

        if qi == 0:
            _zero_buf(hb0, 2)

            def zacc2(j, _):
                pltpu.sync_copy(hb0, acc_sh.at[pl.ds(s * 640 + j * B, B)])
                return 0
            lax.fori_loop(0, 640 // B, zacc2, 0)
            plsc.subcore_barrier()


def _sc2(hg4, src, dst):
    mesh = plsc.VectorSubcoreMesh(core_axis_name="c", subcore_axis_name="s")
    f = functools.partial(
        pl.kernel,
        out_type=[
            jax.ShapeDtypeStruct((NP, 32), f32),
            jax.ShapeDtypeStruct((NP, 32), f32),
            jax.ShapeDtypeStruct((NP, 32), f32),
            jax.ShapeDtypeStruct((NP, 32), f32),
        ],
        mesh=mesh,
        scratch_types=[
            pltpu.VMEM((EPT,), i32),
            pltpu.VMEM((EPT,), i32),
            pltpu.VMEM((B, 32), f32),
            pltpu.VMEM((B, 32), f32),
            pltpu.VMEM((B, 32), f32),
            pltpu.VMEM((B,), i32),
            pltpu.VMEM((B,), i32),
            pltpu.VMEM((B,), i32),
            pltpu.VMEM((B,), i32),
            pltpu.VMEM((B,), i32),
            pltpu.VMEM((B,), i32),
            pltpu.SemaphoreType.DMA,
            pltpu.SemaphoreType.DMA,
            pltpu.SemaphoreType.DMA,
            pltpu.SemaphoreType.DMA,
            pltpu.SemaphoreType.DMA,
            pltpu.SemaphoreType.DMA,
            pltpu.VMEM_SHARED((NP, 32), f32),
        ],
        compiler_params=_SC_PARAMS,
    )(_sc2_body)
    return f(hg4, src, dst)


def kernel(x, edge_index, edge_attr, goal_feat, batch,
           attW, a_src, a_dst, a_edge, gcnW, gcnb, dW1, db1, dW2, db2):
    src = edge_index[0]
    dst = edge_index[1]

    h, hs, hd, eatt = _tc1(x, attW, a_src.reshape(D_IN, 1),
                           a_dst.reshape(D_IN, 1),
                           edge_attr, a_edge.reshape(16, 1))

    o1q0, o1q1, o1q2, o1q3, deg, den = _sc1(h.reshape(4 * N, 64),
                                            hs.reshape(N), hd.reshape(N),
                                            eatt, src, dst)

    hg, hgs, dinv = _tc2([o1q0, o1q1, o1q2, o1q3],
                         den.reshape(NP)[:N].reshape(N, 1),
                         deg.reshape(NP)[:N].reshape(N, 1), gcnW)

    o2q = _sc2(hgs.reshape(4 * N, 32), src, dst)

    pred, x2 = _tc3(o2q, hg, dinv, goal_feat,
                    gcnb.reshape(1, 128), dW1[:128], dW1[128:],
                    db1.reshape(1, D_IN), dW2, db2.reshape(1, 1))
    return (pred, x2)


# speedup vs baseline: 1.0097x; 1.0097x over previous
import functools

import jax
import jax.numpy as jnp
from jax import lax
from jax.experimental import pallas as pl
from jax.experimental.pallas import tpu as pltpu
from jax.experimental.pallas import tpu_sc as plsc

N = 10000
E = 160000
D_IN = 256
NP = 10240
NR = NP // 16
NC, NS, L = 2, 16, 16
B = 80
EPT = E // NS
NB = EPT // B
f32 = jnp.float32
i32 = jnp.int32

_SC_PARAMS = pltpu.CompilerParams(use_tc_tiling_on_sc=False,
                                  needs_layout_passes=False)


RB = 1000


RE = E // (N // RB)


def _tc1_body(x_r, w_r, as_r, ad_r, ea_r, ae_r, h_r, hs_r, hd_r, eo_r):
    i = pl.program_id(0)
    hb = jnp.dot(x_r[...], w_r[...], preferred_element_type=f32)
    h_r[...] = hb
    hs_r[...] = jnp.dot(hb, as_r[...], preferred_element_type=f32)
    hd_r[...] = jnp.dot(hb, ad_r[...], preferred_element_type=f32)
    eo_r[pl.ds(i * RE, RE)] = jnp.dot(ea_r[...], ae_r[...],
                                      preferred_element_type=f32)[:, 0]


def _tc1(x, attW, a_src2, a_dst2, edge_attr, a_edge2):
    return pl.pallas_call(
        _tc1_body,
        grid=(N // RB,),
        in_specs=[
            pl.BlockSpec((RB, D_IN), lambda i: (i, 0)),
            pl.BlockSpec((D_IN, D_IN), lambda i: (0, 0)),
            pl.BlockSpec((D_IN, 1), lambda i: (0, 0)),
            pl.BlockSpec((D_IN, 1), lambda i: (0, 0)),
            pl.BlockSpec((RE, 16), lambda i: (i, 0)),
            pl.BlockSpec((16, 1), lambda i: (0, 0)),
        ],
        out_specs=[
            pl.BlockSpec((RB, D_IN), lambda i: (i, 0)),
            pl.BlockSpec((RB, 1), lambda i: (i, 0)),
            pl.BlockSpec((RB, 1), lambda i: (i, 0)),
            pl.BlockSpec((E,), lambda i: (0,)),
        ],
        out_shape=[
            jax.ShapeDtypeStruct((N, D_IN), f32),
            jax.ShapeDtypeStruct((N, 1), f32),
            jax.ShapeDtypeStruct((N, 1), f32),
            jax.ShapeDtypeStruct((E,), f32),
        ],
    )(x, attW, a_src2, a_dst2, edge_attr, a_edge2)


def _tc2_body(o0_r, o1_r, o2_r, o3_r, den_r, deg_r, gw_r, hg_r, hgs_r, dinv_r):
    rden = 1.0 / (den_r[...] + 1e-16)
    x1 = jnp.concatenate(
        [jnp.maximum(o0_r[...] * rden, 0.0), jnp.maximum(o1_r[...] * rden, 0.0),
         jnp.maximum(o2_r[...] * rden, 0.0), jnp.maximum(o3_r[...] * rden, 0.0)],
        axis=1)
    acc = jnp.dot(x1, gw_r[...], preferred_element_type=f32)
    hg_r[...] = acc
    dv = lax.rsqrt(deg_r[...] + 1.0)
    dinv_r[...] = dv
    hgs_r[...] = dv * acc


def _tc2(o1q, den, deg, gcnW):
    return pl.pallas_call(
        _tc2_body,
        grid=(N // RB,),
        in_specs=[
            pl.BlockSpec((RB, 64), lambda i: (i, 0)),
            pl.BlockSpec((RB, 64), lambda i: (i, 0)),
            pl.BlockSpec((RB, 64), lambda i: (i, 0)),
            pl.BlockSpec((RB, 64), lambda i: (i, 0)),
            pl.BlockSpec((RB, 1), lambda i: (i, 0)),
            pl.BlockSpec((RB, 1), lambda i: (i, 0)),
            pl.BlockSpec((D_IN, 128), lambda i: (0, 0)),
        ],
        out_specs=[
            pl.BlockSpec((RB, 128), lambda i: (i, 0)),
            pl.BlockSpec((RB, 128), lambda i: (i, 0)),
            pl.BlockSpec((RB, 1), lambda i: (i, 0)),
        ],
        out_shape=[
            jax.ShapeDtypeStruct((N, 128), f32),
            jax.ShapeDtypeStruct((N, 128), f32),
            jax.ShapeDtypeStruct((N, 1), f32),
        ],
    )(o1q[0], o1q[1], o1q[2], o1q[3], den, deg, gcnW)


def _tc3_body(p0_r, p1_r, p2_r, p3_r, hg_r, dinv_r, goal_r, gcnb_r,
              w1a_r, w1b_r, db1_r, w2_r, db2_r, pred_r, x2_r):
    dv = dinv_r[...]
    sc_part = jnp.concatenate(
        [p0_r[...], p1_r[...], p2_r[...], p3_r[...]], axis=1)
    out2 = dv * sc_part + (dv * dv) * hg_r[...] + gcnb_r[...]
    x2 = jnp.maximum(out2, 0.0)
    x2_r[...] = x2
    hdist = (jnp.dot(x2, w1a_r[...], preferred_element_type=f32)
             + jnp.dot(goal_r[...], w1b_r[...], preferred_element_type=f32)
             + db1_r[...])
    hdist = jnp.maximum(hdist, 0.0)
    pred_r[...] = jnp.dot(hdist, w2_r[...], preferred_element_type=f32) + db2_r[...]


def _tc3(p, hg, dinv, goal, gcnb2, w1a, w1b, db1_2, dW2, db2_2):
    return pl.pallas_call(
        _tc3_body,
        grid=(N // RB,),
        in_specs=[
            pl.BlockSpec((RB, 32), lambda i: (i, 0)),
            pl.BlockSpec((RB, 32), lambda i: (i, 0)),
            pl.BlockSpec((RB, 32), lambda i: (i, 0)),
            pl.BlockSpec((RB, 32), lambda i: (i, 0)),
            pl.BlockSpec((RB, 128), lambda i: (i, 0)),
            pl.BlockSpec((RB, 1), lambda i: (i, 0)),
            pl.BlockSpec((RB, D_IN), lambda i: (i, 0)),
            pl.BlockSpec((1, 128), lambda i: (0, 0)),
            pl.BlockSpec((128, D_IN), lambda i: (0, 0)),
            pl.BlockSpec((D_IN, D_IN), lambda i: (0, 0)),
            pl.BlockSpec((1, D_IN), lambda i: (0, 0)),
            pl.BlockSpec((D_IN, 1), lambda i: (0, 0)),
            pl.BlockSpec((1, 1), lambda i: (0, 0)),
        ],
        out_specs=[
            pl.BlockSpec((RB, 1), lambda i: (i, 0)),
            pl.BlockSpec((RB, 128), lambda i: (i, 0)),
        ],
        out_shape=[
            jax.ShapeDtypeStruct((N, 1), f32),
            jax.ShapeDtypeStruct((N, 128), f32),
        ],
    )(p[0], p[1], p[2], p[3], hg, dinv, goal, gcnb2, w1a, w1b, db1_2,
      dW2, db2_2)


def _zero_2d(ref, nrows):
    z = jnp.zeros((L,), f32)

    def body(r, _):
        ref[r] = z
        return 0
    lax.fori_loop(0, nrows, body, 0)


def _zero_buf(buf, w):
    z = jnp.zeros((L,), f32)

    def body(b, _):
        for f in range(w):
            buf[b, pl.ds(f * L, L)] = z
        return 0
    lax.fori_loop(0, B, body, 0)


def _scale_rows(buf, w, wref, woff):
    def body(b, _):
        spl = plsc.load_gather(wref, [jnp.full((L,), woff + b, i32)])
        for f in range(w):
            sl = pl.ds(f * L, L)
            buf[b, sl] = buf[b, sl] * spl
        return 0
    lax.fori_loop(0, B, body, 0)


def _iota_fill(ixb, base):
    io = lax.iota(i32, L)

    def body(j, _):
        ixb[pl.ds(j * L, L)] = io + (base + j * L)
        return 0
    lax.fori_loop(0, B // L, body, 0)


def _tile_reduce(part_v, sh, ixb):
    def body(j, _):
        _iota_fill(ixb, j * B)
        pltpu.sync_copy(part_v.at[pl.ds(j * B, B)], sh.at[ixb], add=True)
        return 0
    lax.fori_loop(0, NR // B, body, 0)


def _spmm_pass(table_h, tq, src_t, wfill, wref, woff_fn, acc_sh, w,
               hbufs, ixs, dbs, gsem, ssem):
    def issue_gather(b, cur):
        off = b * B

        def wl(j, _):
            sl = pl.ds(j * L, L)
            ixs[cur][sl] = src_t[pl.ds(off + j * L, L)] * 4 + tq
            return 0
        lax.fori_loop(0, B // L, wl, 0)
        pltpu.async_copy(table_h.at[ixs[cur]], hbufs[cur], gsem[cur])

    def wait_gather(cur):
        pltpu.make_async_copy(table_h.at[ixs[cur]], hbufs[cur],
                              gsem[cur]).wait()

    def issue_scatter(b, cur):
        wfill(b, cur)
        if wref is not None:
            _scale_rows(hbufs[cur], w, wref, woff_fn(b))
        pltpu.async_copy(hbufs[cur], acc_sh.at[dbs[cur]], ssem[cur], add=True)

    def wait_scatter(cur):
        pltpu.make_async_copy(hbufs[cur], acc_sh.at[dbs[cur]], ssem[cur]).wait()

    issue_gather(0, 0)
    issue_gather(1, 1)
    wait_gather(0)
    issue_gather(2, 2)
    issue_scatter(0, 0)
    wait_gather(1)
    issue_scatter(1, 1)

    def group(g, _):
        b = g * 3 + 2
        for j, cur in enumerate((2, 0, 1)):
            nxt = (cur + 1) % 3
            wait_scatter(nxt)
            issue_gather(b + j + 1, nxt)
            wait_gather(cur)
            issue_scatter(b + j, cur)
        return 0
    lax.fori_loop(0, (NB - 5) // 3, group, 0)

    wait_scatter(0)
    issue_gather(NB - 2, 0)
    wait_gather(2)
    issue_scatter(NB - 3, 2)
    wait_scatter(1)
    issue_gather(NB - 1, 1)
    wait_gather(0)
    issue_scatter(NB - 2, 0)
    wait_gather(1)
    issue_scatter(NB - 1, 1)
    wait_scatter(2)
    wait_scatter(0)
    wait_scatter(1)


def _sc1_body(h4_h, hs_h, hd_h, ea_h, ei_h,
              o0_h, o1_h, o2_h, o3_h, deg_h, den_h,
              src_t, dst_t, ea_t, hs_v, hd_v, den_v, deg_v, zbuf,
              hb0, hb1, hb2, ix0, ix1, ix2, db0, db1_, db2_,
              gs0, gs1, gs2, ss0, ss1, ss2,
              acc_sh, den_sh, deg_sh):
    c = lax.axis_index("c")
    s = lax.axis_index("s")
    hbufs = [hb0, hb1, hb2]
    ixs = [ix0, ix1, ix2]
    dbs = [db0, db1_, db2_]
    gsem = [gs0, gs1, gs2]
    ssem = [ss0, ss1, ss2]
    o_hs = [o0_h, o1_h, o2_h, o3_h]
    base = s * EPT

    pltpu.sync_copy(ei_h.at[0, pl.ds(base, EPT)], src_t)
    pltpu.sync_copy(ei_h.at[1, pl.ds(base, EPT)], dst_t)
    pltpu.sync_copy(ea_h.at[pl.ds(base, EPT)], ea_t)
    pltpu.sync_copy(hs_h, hs_v)
    pltpu.sync_copy(hd_h, hd_v)

    _zero_2d(den_v, NR)
    _zero_2d(deg_v, NR)
    _zero_2d(zbuf, NR // NS)

    srows = pl.ds(s * (NR // NS), NR // NS)
    pltpu.sync_copy(zbuf, den_sh.at[srows])
    pltpu.sync_copy(zbuf, deg_sh.at[srows])
    _zero_buf(hb0, 4)

    def zacc(j, _):
        pltpu.sync_copy(hb0, acc_sh.at[pl.ds(s * 640 + j * B, B)])
        return 0
    lax.fori_loop(0, 640 // B, zacc, 0)

    ones = jnp.ones((L,), f32)

    def p1(k, _):
        off = k * L
        s16 = src_t[pl.ds(off, L)]
        d16 = dst_t[pl.ds(off, L)]
        z = (plsc.load_gather(hs_v, [s16]) + plsc.load_gather(hd_v, [d16])
             + ea_t[pl.ds(off, L)])
        zl = jnp.where(z >= 0.0, z, 0.2 * z)
        e16 = jnp.exp(zl)
        ea_t[pl.ds(off, L)] = e16
        dr = lax.shift_right_logical(d16, 4)
        dc = lax.bitwise_and(d16, 15)
        plsc.addupdate_scatter(den_v, [dr, dc], e16)
        plsc.addupdate_scatter(deg_v, [dr, dc], ones)
        return 0
    lax.fori_loop(0, EPT // L, p1, 0)

    plsc.subcore_barrier()
    _tile_reduce(den_v, den_sh, ix0)
    _tile_reduce(deg_v, deg_sh, ix0)
    plsc.subcore_barrier()

    @pl.when(c == 0)
    def _():
        pltpu.sync_copy(deg_sh.at[srows], deg_h.at[srows])
        pltpu.sync_copy(den_sh.at[srows], den_h.at[srows])

    def wfill(b, cur):
        off = b * B

        def w(j, _):
            sl = pl.ds(j * L, L)
            dbs[cur][sl] = dst_t[pl.ds(off + j * L, L)]
            return 0
        lax.fori_loop(0, B // L, w, 0)

    for qi in range(2):
        tq = 2 * c + qi
        _spmm_pass(h4_h, tq, src_t, wfill, ea_t, lambda b: b * B,
                   acc_sh, 4, hbufs, ixs, dbs, gsem, ssem)
        plsc.subcore_barrier()
        rows = pl.ds(s * 640, 640)

        @pl.when(c == 0)
        def _():
            pltpu.sync_copy(acc_sh.at[rows], o_hs[qi].at[rows])

        @pl.when(c == 1)
        def _():
            pltpu.sync_copy(acc_sh.at[rows], o_hs[2 + qi].at[rows])

        if qi == 0:
            _zero_buf(hb0, 4)

            def zacc2(j, _):
                pltpu.sync_copy(hb0, acc_sh.at[pl.ds(s * 640 + j * B, B)])
                return 0
            lax.fori_loop(0, 640 // B, zacc2, 0)
            plsc.subcore_barrier()


def _sc1(h4, hs, hd, eatt, edge_index):
    mesh = plsc.VectorSubcoreMesh(core_axis_name="c", subcore_axis_name="s")
    f = functools.partial(
        pl.kernel,
        out_type=[
            jax.ShapeDtypeStruct((NP, 64), f32),
            jax.ShapeDtypeStruct((NP, 64), f32),
            jax.ShapeDtypeStruct((NP, 64), f32),
            jax.ShapeDtypeStruct((NP, 64), f32),
            jax.ShapeDtypeStruct((NR, 16), f32),
            jax.ShapeDtypeStruct((NR, 16), f32),
        ],
        mesh=mesh,
        scratch_types=[
            pltpu.VMEM((EPT,), i32),
            pltpu.VMEM((EPT,), i32),
            pltpu.VMEM((EPT,), f32),
            pltpu.VMEM((N,), f32),
            pltpu.VMEM((N,), f32),
            pltpu.VMEM((NR, 16), f32),
            pltpu.VMEM((NR, 16), f32),
            pltpu.VMEM((NR // NS, 16), f32),
            pltpu.VMEM((B, 64), f32),
            pltpu.VMEM((B, 64), f32),
            pltpu.VMEM((B, 64), f32),
            pltpu.VMEM((B,), i32),
            pltpu.VMEM((B,), i32),
            pltpu.VMEM((B,), i32),
            pltpu.VMEM((B,), i32),
            pltpu.VMEM((B,), i32),
            pltpu.VMEM((B,), i32),
            pltpu.SemaphoreType.DMA,
            pltpu.SemaphoreType.DMA,
            pltpu.SemaphoreType.DMA,
            pltpu.SemaphoreType.DMA,
            pltpu.SemaphoreType.DMA,
            pltpu.SemaphoreType.DMA,
            pltpu.VMEM_SHARED((NP, 64), f32),
            pltpu.VMEM_SHARED((NR, 16), f32),
            pltpu.VMEM_SHARED((NR, 16), f32),
        ],
        compiler_params=_SC_PARAMS,
    )(_sc1_body)
    return f(h4, hs, hd, eatt, edge_index)


def _sc2_body(hg4_h, ei_h,
              o0_h, o1_h, o2_h, o3_h,
              src_t, dst_t,
              hb0, hb1, hb2, ix0, ix1, ix2, db0, db1_, db2_,
              gs0, gs1, gs2, ss0, ss1, ss2,
              acc_sh):
    c = lax.axis_index("c")
    s = lax.axis_index("s")
    hbufs = [hb0, hb1, hb2]
    ixs = [ix0, ix1, ix2]
    dbs = [db0, db1_, db2_]
    gsem = [gs0, gs1, gs2]
    ssem = [ss0, ss1, ss2]
    o_hs = [o0_h, o1_h, o2_h, o3_h]
    base = s * EPT

    pltpu.sync_copy(ei_h.at[0, pl.ds(base, EPT)], src_t)
    pltpu.sync_copy(ei_h.at[1, pl.ds(base, EPT)], dst_t)

    _zero_buf(hb0, 2)

    def zacc(j, _):
        pltpu.sync_copy(hb0, acc_sh.at[pl.ds(s * 640 + j * B, B)])
        return 0
    lax.fori_loop(0, 640 // B, zacc, 0)
    plsc.subcore_barrier()

    def wfill(b, cur):
        off = b * B

        def w(j, _):
            sl = pl.ds(j * L, L)
            dbs[cur][sl] = dst_t[pl.ds(off + j * L, L)]
            return 0
        lax.fori_loop(0, B // L, w, 0)

    for qi in range(2):
        tq = 2 * c + qi
        _spmm_pass(hg4_h, tq, src_t, wfill, None, lambda b: b * B,
                   acc_sh, 2, hbufs, ixs, dbs, gsem, ssem)
        plsc.subcore_barrier()
        rows = pl.ds(s * 640, 640)

        @pl.when(c == 0)
        def _():
            pltpu.sync_copy(acc_sh.at[rows], o_hs[qi].at[rows])

        @pl.when(c == 1)
        def _():
            pltpu.sync_copy(acc_sh.at[rows], o_hs[2 + qi].at[rows])

        if qi == 0:
            _zero_buf(hb0, 2)

            def zacc2(j, _):
                pltpu.sync_copy(hb0, acc_sh.at[pl.ds(s * 640 + j * B, B)])
                return 0
            lax.fori_loop(0, 640 // B, zacc2, 0)
            plsc.subcore_barrier()


def _sc2(hg4, edge_index):
    mesh = plsc.VectorSubcoreMesh(core_axis_name="c", subcore_axis_name="s")
    f = functools.partial(
        pl.kernel,
        out_type=[
            jax.ShapeDtypeStruct((NP, 32), f32),
            jax.ShapeDtypeStruct((NP, 32), f32),
            jax.ShapeDtypeStruct((NP, 32), f32),
            jax.ShapeDtypeStruct((NP, 32), f32),
        ],
        mesh=mesh,
        scratch_types=[
            pltpu.VMEM((EPT,), i32),
            pltpu.VMEM((EPT,), i32),
            pltpu.VMEM((B, 32), f32),
            pltpu.VMEM((B, 32), f32),
            pltpu.VMEM((B, 32), f32),
            pltpu.VMEM((B,), i32),
            pltpu.VMEM((B,), i32),
            pltpu.VMEM((B,), i32),
            pltpu.VMEM((B,), i32),
            pltpu.VMEM((B,), i32),
            pltpu.VMEM((B,), i32),
            pltpu.SemaphoreType.DMA,
            pltpu.SemaphoreType.DMA,
            pltpu.SemaphoreType.DMA,
            pltpu.SemaphoreType.DMA,
            pltpu.SemaphoreType.DMA,
            pltpu.SemaphoreType.DMA,
            pltpu.VMEM_SHARED((NP, 32), f32),
        ],
        compiler_params=_SC_PARAMS,
    )(_sc2_body)
    return f(hg4, edge_index)


def kernel(x, edge_index, edge_attr, goal_feat, batch,
           attW, a_src, a_dst, a_edge, gcnW, gcnb, dW1, db1, dW2, db2):
    h, hs, hd, eatt = _tc1(x, attW, a_src.reshape(D_IN, 1),
                           a_dst.reshape(D_IN, 1),
                           edge_attr, a_edge.reshape(16, 1))

    o1q0, o1q1, o1q2, o1q3, deg, den = _sc1(h.reshape(4 * N, 64),
                                            hs.reshape(N), hd.reshape(N),
                                            eatt, edge_index)

    hg, hgs, dinv = _tc2([o1q0, o1q1, o1q2, o1q3],
                         den.reshape(NP)[:N].reshape(N, 1),
                         deg.reshape(NP)[:N].reshape(N, 1), gcnW)

    o2q = _sc2(hgs.reshape(4 * N, 32), edge_index)

    pred, x2 = _tc3(o2q, hg, dinv, goal_feat,
                    gcnb.reshape(1, 128), dW1[:128], dW1[128:],
                    db1.reshape(1, D_IN), dW2, db2.reshape(1, 1))
    return (pred, x2)
